# baseline (speedup 1.0000x reference)
import functools

import jax
import jax.numpy as jnp
from jax import lax
from jax.experimental import pallas as pl
from jax.experimental.pallas import tpu as pltpu
from jax.experimental.pallas import tpu_sc as plsc

_B = 16384
_C = 1000
_K = 64
_LN2 = 0.6931471805599453


def _plog16(x):
    xi = lax.bitcast_convert_type(x, jnp.int32)
    ef = lax.shift_right_logical(xi, 23).astype(jnp.float32)
    mi = (xi & 0x007FFFFF) | 0x3F800000
    m = lax.bitcast_convert_type(mi, jnp.float32)
    s = (m - 1.0) / (m + 1.0)
    z = s * s
    p = z * (1.0 / 7.0) + (1.0 / 5.0)
    p = p * z + (1.0 / 3.0)
    p = p * z + 1.0
    return (ef * _LN2 + (s + s) * p) - (127.0 * _LN2)


_GATHER_DNUMS = lax.GatherDimensionNumbers(
    offset_dims=(), collapsed_slice_dims=(0,), start_index_map=(0,))


def _shuffle_xor(x, k):
    idx = lax.iota(jnp.int32, 16) ^ k
    return lax.gather(x, idx[:, None], dimension_numbers=_GATHER_DNUMS,
                      slice_sizes=(1,),
                      mode=lax.GatherScatterMode.PROMISE_IN_BOUNDS)


def _all_reduce16(x, op):
    for k in (1, 2, 4, 8):
        x = op(x, _shuffle_xor(x, k))
    return x


def _sc_partials(score, targets):
    info = plsc.get_sparse_core_info()
    NC, NS, L = info.num_cores, info.num_subcores, info.num_lanes
    NW = NC * NS
    RPW = _B // NW
    RB = 16
    NCH = RPW // RB
    NFULL = _C // L
    AW = (NFULL + 1) * L
    BW = RB * _C + L
    mesh = plsc.VectorSubcoreMesh(core_axis_name="c", subcore_axis_name="s")

    @functools.partial(
        pl.kernel,
        mesh=mesh,
        out_type=jax.ShapeDtypeStruct((NW, _K, AW), jnp.float32),
        scratch_types=[
            pltpu.VMEM((RPW,), jnp.int32),
            pltpu.VMEM((BW,), jnp.float32),
            pltpu.VMEM((BW,), jnp.float32),
            pltpu.VMEM((_K, AW), jnp.float32),
            pltpu.SemaphoreType.DMA,
            pltpu.SemaphoreType.DMA,
            pltpu.SemaphoreType.DMA,
        ],
    )
    def k(score_h, tgt_h, out_h, tgt_v, buf0, buf1, acc, sem0, sem1, tsem):
        wid = lax.axis_index("s") * NC + lax.axis_index("c")
        row0 = wid * RPW
        pltpu.async_copy(tgt_h.at[pl.ds(row0, RPW)], tgt_v, tsem).wait()

        zero = jnp.zeros((L,), jnp.float32)

        def zbody(i, carry):
            def zrow(c, carry2):
                acc[i, pl.ds(c * L, L)] = zero
                return carry2
            lax.fori_loop(0, NFULL + 1, zrow, 0, unroll=9)
            return carry

        lax.fori_loop(0, _K, zbody, 0)
        iota = lax.iota(jnp.int32, L)

        def process(buf, g):
            tvec = tgt_v[pl.ds(g * RB, L)]

            @plsc.parallel_loop(0, RB, unroll=2)
            def row_body(r):
                tsel = jnp.where(iota == r, tvec, 0)
                t = _all_reduce16(tsel, jnp.add)[0]
                roff = r * _C

                @plsc.parallel_loop(
                    0, NFULL, carry=jnp.full((L,), -1.0, jnp.float32),
                    unroll=16)
                def p1out(c, vmax):
                    v = buf[pl.ds(roff + c * L, L)]
                    return jnp.maximum(vmax, v)

                vmax = p1out
                vt = buf[pl.ds(roff + NFULL * L, L)]
                colt = NFULL * L + iota
                maskt = colt < _C
                vmax = jnp.maximum(vmax, jnp.where(maskt, vt, -1.0))
                mxv = _all_reduce16(vmax, jnp.maximum)

                tsplat = jnp.zeros((L,), jnp.int32) + t
                tch = buf[pl.ds(roff + ((t >> 4) << 4), L)]
                s_tv = _all_reduce16(
                    jnp.where(iota == (t & 15), tch, 0.0), jnp.add)
                hit = jnp.zeros((L,), jnp.int32)
                for cc in range(4):
                    vv = buf[pl.ds(roff + cc * L, L)]
                    colv = cc * L + iota
                    hit = hit + jnp.where(
                        (vv == mxv) & (colv < tsplat), 1, 0)
                hitsum = _all_reduce16(hit, jnp.add)
                w = jnp.where((s_tv == mxv) & (hitsum == 0), s_tv,
                              jnp.float32(1.0))

                @plsc.parallel_loop(0, NFULL, unroll=12)
                def _(c):
                    v = buf[pl.ds(roff + c * L, L)]
                    gval = w * _plog16(v + 1e-12)
                    plsc.addupdate(acc.at[t, pl.ds(c * L, L)], gval)
                vt2 = buf[pl.ds(roff + NFULL * L, L)]
                gt = w * _plog16(vt2 + 1e-12)
                gt = jnp.where(maskt, gt, 0.0)
                plsc.addupdate(acc.at[t, pl.ds(NFULL * L, L)], gt)

        pltpu.async_copy(score_h.at[pl.ds(row0 * _C, RB * _C)],
                         buf0.at[pl.ds(0, RB * _C)], sem0)

        def outer(gg, carry):
            for b in range(2):
                g = 2 * gg + b
                buf, sem = (buf0, sem0) if b == 0 else (buf1, sem1)
                nbuf, nsem = (buf1, sem1) if b == 0 else (buf0, sem0)

                @pl.when(g + 1 < NCH)
                def _():
                    pltpu.async_copy(
                        score_h.at[pl.ds((row0 + (g + 1) * RB) * _C, RB * _C)],
                        nbuf.at[pl.ds(0, RB * _C)], nsem)

                pltpu.make_async_copy(
                    score_h.at[pl.ds(0, RB * _C)],
                    buf.at[pl.ds(0, RB * _C)], sem).wait()
                process(buf, g)
            return carry

        lax.fori_loop(0, NCH // 2, outer, 0)
        pltpu.sync_copy(acc, out_h.at[wid])

    return k(score.reshape(-1), targets)


def _finalize(partials, tg2d):
    def body(p_ref, t_ref, o_ref):
        M = jnp.sum(p_ref[...], axis=0)[:, :_C]
        tg = t_ref[...]
        ks = lax.broadcasted_iota(jnp.int32, (_K,) + tg.shape, 0)
        cnt = jnp.sum((tg[None] == ks).astype(jnp.float32), axis=(1, 2))
        dmask = (lax.broadcasted_iota(jnp.int32, (_K, _C), 0)
                 == lax.broadcasted_iota(jnp.int32, (_K, _C), 1))
        dg = jnp.sum(jnp.where(dmask, M, 0.0), axis=1)
        rs = jnp.sum(M, axis=1) - dg
        ss = jnp.sum(M * M, axis=1) - dg * dg
        n = jnp.float32(_C - 1)
        var = (ss - rs * rs / n) / (n - 1.0)
        std = jnp.sqrt(jnp.maximum(var, 0.0))
        present = cnt > 0.5
        kf = jnp.sum(present.astype(jnp.float32))
        invc = jnp.where(present, 1.0 / jnp.maximum(cnt, 1.0), 0.0)
        loss = jnp.sum(invc) * jnp.sum(jnp.where(present, std, 0.0)) / kf
        o_ref[...] = jnp.broadcast_to(loss, (1, 1))

    out = pl.pallas_call(
        body, out_shape=jax.ShapeDtypeStruct((1, 1), jnp.float32))(
            partials, tg2d)
    return out[0, 0]


def kernel(substitute_score, targets):
    parts = _sc_partials(substitute_score, targets)
    tg2d = targets.reshape(128, 128)
    return _finalize(parts, tg2d)


# speedup vs baseline: 1.0166x; 1.0166x over previous
import functools

import jax
import jax.numpy as jnp
from jax import lax
from jax.experimental import pallas as pl
from jax.experimental.pallas import tpu as pltpu
from jax.experimental.pallas import tpu_sc as plsc

_B = 16384
_C = 1000
_K = 64
_LN2 = 0.6931471805599453


def _plog16(x):
    xi = lax.bitcast_convert_type(x, jnp.int32)
    ef = lax.shift_right_logical(xi, 23).astype(jnp.float32)
    mi = (xi & 0x007FFFFF) | 0x3F800000
    m = lax.bitcast_convert_type(mi, jnp.float32)
    s = (m - 1.0) / (m + 1.0)
    z = s * s
    p = z * (1.0 / 7.0) + (1.0 / 5.0)
    p = p * z + (1.0 / 3.0)
    p = p * z + 1.0
    return (ef * _LN2 + (s + s) * p) - (127.0 * _LN2)


_GATHER_DNUMS = lax.GatherDimensionNumbers(
    offset_dims=(), collapsed_slice_dims=(0,), start_index_map=(0,))


def _shuffle_xor(x, k):
    idx = lax.iota(jnp.int32, 16) ^ k
    return lax.gather(x, idx[:, None], dimension_numbers=_GATHER_DNUMS,
                      slice_sizes=(1,),
                      mode=lax.GatherScatterMode.PROMISE_IN_BOUNDS)


def _all_reduce16(x, op):
    for k in (1, 2, 4, 8):
        x = op(x, _shuffle_xor(x, k))
    return x


def _sc_partials(score, targets):
    info = plsc.get_sparse_core_info()
    NC, NS, L = info.num_cores, info.num_subcores, info.num_lanes
    NW = NC * NS
    RPW = _B // NW
    RB = 16
    NCH = RPW // RB
    NFULL = _C // L
    AW = (NFULL + 1) * L
    BW = RB * _C + L
    mesh = plsc.VectorSubcoreMesh(core_axis_name="c", subcore_axis_name="s")

    @functools.partial(
        pl.kernel,
        mesh=mesh,
        out_type=jax.ShapeDtypeStruct((NW, _K, AW), jnp.float32),
        scratch_types=[
            pltpu.VMEM((RPW,), jnp.int32),
            pltpu.VMEM((BW,), jnp.float32),
            pltpu.VMEM((BW,), jnp.float32),
            pltpu.VMEM((_K, AW), jnp.float32),
            pltpu.SemaphoreType.DMA,
            pltpu.SemaphoreType.DMA,
            pltpu.SemaphoreType.DMA,
        ],
    )
    def k(score_h, tgt_h, out_h, tgt_v, buf0, buf1, acc, sem0, sem1, tsem):
        wid = lax.axis_index("s") * NC + lax.axis_index("c")
        row0 = wid * RPW
        pltpu.async_copy(tgt_h.at[pl.ds(row0, RPW)], tgt_v, tsem).wait()

        zero = jnp.zeros((L,), jnp.float32)

        def zbody(i, carry):
            def zrow(c, carry2):
                acc[i, pl.ds(c * L, L)] = zero
                return carry2
            lax.fori_loop(0, NFULL + 1, zrow, 0, unroll=9)
            return carry

        lax.fori_loop(0, _K, zbody, 0)
        iota = lax.iota(jnp.int32, L)

        def process(buf, g):
            tvec = tgt_v[pl.ds(g * RB, L)]

            @plsc.parallel_loop(0, RB, unroll=2)
            def row_body(r):
                tsel = jnp.where(iota == r, tvec, 0)
                t = _all_reduce16(tsel, jnp.add)[0]
                roff = r * _C

                @plsc.parallel_loop(
                    0, NFULL, carry=jnp.full((L,), -1.0, jnp.float32),
                    unroll=16)
                def p1out(c, vmax):
                    v = buf[pl.ds(roff + c * L, L)]
                    return jnp.maximum(vmax, v)

                vmax = p1out
                vt = buf[pl.ds(roff + NFULL * L, L)]
                colt = NFULL * L + iota
                maskt = colt < _C
                vmax = jnp.maximum(vmax, jnp.where(maskt, vt, -1.0))
                mxv = _all_reduce16(vmax, jnp.maximum)

                tsplat = jnp.zeros((L,), jnp.int32) + t
                tch = buf[pl.ds(roff + ((t >> 4) << 4), L)]
                s_tv = _all_reduce16(
                    jnp.where(iota == (t & 15), tch, 0.0), jnp.add)
                hit = jnp.zeros((L,), jnp.int32)
                for cc in range(4):
                    vv = buf[pl.ds(roff + cc * L, L)]
                    colv = cc * L + iota
                    hit = hit + jnp.where(
                        (vv == mxv) & (colv < tsplat), 1, 0)
                hitsum = _all_reduce16(hit, jnp.add)
                w = jnp.where((s_tv == mxv) & (hitsum == 0), s_tv,
                              jnp.float32(1.0))

                @plsc.parallel_loop(0, NFULL, unroll=10)
                def _(c):
                    v = buf[pl.ds(roff + c * L, L)]
                    gval = w * _plog16(v + 1e-12)
                    plsc.addupdate(acc.at[t, pl.ds(c * L, L)], gval)
                vt2 = buf[pl.ds(roff + NFULL * L, L)]
                gt = w * _plog16(vt2 + 1e-12)
                gt = jnp.where(maskt, gt, 0.0)
                plsc.addupdate(acc.at[t, pl.ds(NFULL * L, L)], gt)

        pltpu.async_copy(score_h.at[pl.ds(row0 * _C, RB * _C)],
                         buf0.at[pl.ds(0, RB * _C)], sem0)

        def outer(gg, carry):
            for b in range(2):
                g = 2 * gg + b
                buf, sem = (buf0, sem0) if b == 0 else (buf1, sem1)
                nbuf, nsem = (buf1, sem1) if b == 0 else (buf0, sem0)

                @pl.when(g + 1 < NCH)
                def _():
                    pltpu.async_copy(
                        score_h.at[pl.ds((row0 + (g + 1) * RB) * _C, RB * _C)],
                        nbuf.at[pl.ds(0, RB * _C)], nsem)

                pltpu.make_async_copy(
                    score_h.at[pl.ds(0, RB * _C)],
                    buf.at[pl.ds(0, RB * _C)], sem).wait()
                process(buf, g)
            return carry

        lax.fori_loop(0, NCH // 2, outer, 0)
        pltpu.sync_copy(acc, out_h.at[wid])

    return k(score.reshape(-1), targets)


def _finalize(partials, tg2d):
    def body(p_ref, t_ref, o_ref):
        M = jnp.sum(p_ref[...], axis=0)[:, :_C]
        tg = t_ref[...]
        ks = lax.broadcasted_iota(jnp.int32, (_K,) + tg.shape, 0)
        cnt = jnp.sum((tg[None] == ks).astype(jnp.float32), axis=(1, 2))
        dmask = (lax.broadcasted_iota(jnp.int32, (_K, _C), 0)
                 == lax.broadcasted_iota(jnp.int32, (_K, _C), 1))
        dg = jnp.sum(jnp.where(dmask, M, 0.0), axis=1)
        rs = jnp.sum(M, axis=1) - dg
        ss = jnp.sum(M * M, axis=1) - dg * dg
        n = jnp.float32(_C - 1)
        var = (ss - rs * rs / n) / (n - 1.0)
        std = jnp.sqrt(jnp.maximum(var, 0.0))
        present = cnt > 0.5
        kf = jnp.sum(present.astype(jnp.float32))
        invc = jnp.where(present, 1.0 / jnp.maximum(cnt, 1.0), 0.0)
        loss = jnp.sum(invc) * jnp.sum(jnp.where(present, std, 0.0)) / kf
        o_ref[...] = jnp.broadcast_to(loss, (1, 1))

    out = pl.pallas_call(
        body, out_shape=jax.ShapeDtypeStruct((1, 1), jnp.float32))(
            partials, tg2d)
    return out[0, 0]


def kernel(substitute_score, targets):
    parts = _sc_partials(substitute_score, targets)
    tg2d = targets.reshape(128, 128)
    return _finalize(parts, tg2d)
